# baseline (speedup 1.0000x reference)
import functools

import jax
import jax.numpy as jnp
from jax.experimental import pallas as pl
from jax.experimental.pallas import tpu as pltpu

_TOP_K = 8


def _gate_kernel(x_ref, w0_ref, b0_ref, w1_ref, b1_ref, out_ref,
                 macc, sacc, *, n_steps, hw):
    i = pl.program_id(0)
    xt = x_ref[...]
    m = jnp.max(xt, axis=0)
    s = jnp.sum(xt, axis=0)

    @pl.when(i == 0)
    def _():
        macc[...] = m
        sacc[...] = s

    @pl.when(i > 0)
    def _():
        macc[...] = jnp.maximum(macc[...], m)
        sacc[...] = sacc[...] + s

    @pl.when(i == n_steps - 1)
    def _():
        pooled = macc[...] + sacc[...] * (1.0 / hw)

        dn = (((1,), (1,)), ((), ()))
        h = jax.lax.dot_general(pooled, w1_ref[...], dn,
                                preferred_element_type=jnp.float32) + b1_ref[...]
        h = jnp.where(h >= 0, h, 0.2 * h)

        z = jax.lax.dot_general(pooled, w0_ref[...], dn,
                                preferred_element_type=jnp.float32) + b0_ref[...]
        noise = jnp.maximum(z, 0.0) + jnp.log1p(jnp.exp(-jnp.abs(z)))

        e = noise.shape[1]
        nmean = jnp.mean(noise, axis=1, keepdims=True)
        d = noise - nmean
        var = jnp.sum(d * d, axis=1, keepdims=True) * (1.0 / (e - 1))
        std = jnp.sqrt(var)
        std = jnp.where(std == 0, 1.0, std)
        scores = h + d / std

        iota = jax.lax.broadcasted_iota(jnp.int32, scores.shape, 1)
        work = scores
        mask = jnp.zeros_like(scores, dtype=jnp.bool_)
        for _ in range(_TOP_K):
            mx = jnp.max(work, axis=1, keepdims=True)
            first = jnp.min(jnp.where(work == mx, iota, e), axis=1, keepdims=True)
            sel = iota == first
            mask = jnp.logical_or(mask, sel)
            work = jnp.where(sel, -1e30, work)

        h_masked = jnp.where(mask, h, -1e9)
        hm = jnp.max(h_masked, axis=1, keepdims=True)
        ex = jnp.exp(h_masked - hm)
        out_ref[...] = ex / jnp.sum(ex, axis=1, keepdims=True)


@functools.partial(jax.jit, static_argnames=("interpret",))
def kernel(x, W0, b0, W1, b1, interpret=False):
    B, C, H, W = x.shape
    E = W0.shape[0]
    hw = H * W
    hw_blk = 28
    n_steps = hw // hw_blk

    xp = x.transpose(2, 3, 0, 1).reshape(hw, B, C)

    out = pl.pallas_call(
        functools.partial(_gate_kernel, n_steps=n_steps, hw=float(hw)),
        grid=(n_steps,),
        in_specs=[
            pl.BlockSpec((hw_blk, B, C), lambda i: (i, 0, 0)),
            pl.BlockSpec((E, C), lambda i: (0, 0)),
            pl.BlockSpec((1, E), lambda i: (0, 0)),
            pl.BlockSpec((E, C), lambda i: (0, 0)),
            pl.BlockSpec((1, E), lambda i: (0, 0)),
        ],
        out_specs=pl.BlockSpec((B, E), lambda i: (0, 0)),
        out_shape=jax.ShapeDtypeStruct((B, E), jnp.float32),
        scratch_shapes=[
            pltpu.VMEM((B, C), jnp.float32),
            pltpu.VMEM((B, C), jnp.float32),
        ],
        interpret=interpret,
    )(xp, W0, b0.reshape(1, E), W1, b1.reshape(1, E))
    return out


# speedup vs baseline: 1.0253x; 1.0253x over previous
import functools

import jax
import jax.numpy as jnp
from jax.experimental import pallas as pl
from jax.experimental.pallas import tpu as pltpu

_TOP_K = 8


def _gate_kernel(x_ref, w0_ref, b0_ref, w1_ref, b1_ref, out_ref,
                 macc, sacc, *, n_steps, hw):
    i = pl.program_id(0)
    xt = x_ref[...]
    m = jnp.max(xt, axis=0)
    s = jnp.sum(xt, axis=0)

    @pl.when(i == 0)
    def _():
        macc[...] = m
        sacc[...] = s

    @pl.when(i > 0)
    def _():
        macc[...] = jnp.maximum(macc[...], m)
        sacc[...] = sacc[...] + s

    @pl.when(i == n_steps - 1)
    def _():
        pooled = macc[...] + sacc[...] * (1.0 / hw)

        dn = (((1,), (1,)), ((), ()))
        h = jax.lax.dot_general(pooled, w1_ref[...], dn,
                                preferred_element_type=jnp.float32) + b1_ref[...]
        h = jnp.where(h >= 0, h, 0.2 * h)

        z = jax.lax.dot_general(pooled, w0_ref[...], dn,
                                preferred_element_type=jnp.float32) + b0_ref[...]
        noise = jnp.maximum(z, 0.0) + jnp.log1p(jnp.exp(-jnp.abs(z)))

        e = noise.shape[1]
        nmean = jnp.mean(noise, axis=1, keepdims=True)
        d = noise - nmean
        var = jnp.sum(d * d, axis=1, keepdims=True) * (1.0 / (e - 1))
        std = jnp.sqrt(var)
        std = jnp.where(std == 0, 1.0, std)
        scores = h + d / std

        iota = jax.lax.broadcasted_iota(jnp.int32, scores.shape, 1)
        work = scores
        mask = jnp.zeros_like(scores, dtype=jnp.bool_)
        for _ in range(_TOP_K):
            mx = jnp.max(work, axis=1, keepdims=True)
            first = jnp.min(jnp.where(work == mx, iota, e), axis=1, keepdims=True)
            sel = iota == first
            mask = jnp.logical_or(mask, sel)
            work = jnp.where(sel, -1e30, work)

        h_masked = jnp.where(mask, h, -1e9)
        hm = jnp.max(h_masked, axis=1, keepdims=True)
        ex = jnp.exp(h_masked - hm)
        out_ref[...] = ex / jnp.sum(ex, axis=1, keepdims=True)


@functools.partial(jax.jit, static_argnames=("interpret",))
def kernel(x, W0, b0, W1, b1, interpret=False):
    B, C, H, W = x.shape
    E = W0.shape[0]
    hw = H * W
    hw_blk = 7
    n_steps = hw // hw_blk

    xp = x.transpose(2, 3, 0, 1).reshape(hw, B, C)

    out = pl.pallas_call(
        functools.partial(_gate_kernel, n_steps=n_steps, hw=float(hw)),
        grid=(n_steps,),
        in_specs=[
            pl.BlockSpec((hw_blk, B, C), lambda i: (i, 0, 0)),
            pl.BlockSpec((E, C), lambda i: (0, 0)),
            pl.BlockSpec((1, E), lambda i: (0, 0)),
            pl.BlockSpec((E, C), lambda i: (0, 0)),
            pl.BlockSpec((1, E), lambda i: (0, 0)),
        ],
        out_specs=pl.BlockSpec((B, E), lambda i: (0, 0)),
        out_shape=jax.ShapeDtypeStruct((B, E), jnp.float32),
        scratch_shapes=[
            pltpu.VMEM((B, C), jnp.float32),
            pltpu.VMEM((B, C), jnp.float32),
        ],
        interpret=interpret,
    )(xp, W0, b0.reshape(1, E), W1, b1.reshape(1, E))
    return out


# speedup vs baseline: 1.0635x; 1.0372x over previous
import functools

import jax
import jax.numpy as jnp
from jax.experimental import pallas as pl
from jax.experimental.pallas import tpu as pltpu

_TOP_K = 8


def _gate_kernel(x_ref, w0_ref, b0_ref, w1_ref, b1_ref, out_ref,
                 macc, sacc, *, n_steps, hw):
    i = pl.program_id(0)
    xt = x_ref[...]
    m = jnp.max(xt, axis=0)
    s = jnp.sum(xt, axis=0)

    @pl.when(i == 0)
    def _():
        macc[...] = m
        sacc[...] = s

    @pl.when(i > 0)
    def _():
        macc[...] = jnp.maximum(macc[...], m)
        sacc[...] = sacc[...] + s

    @pl.when(i == n_steps - 1)
    def _():
        pooled = macc[...] + sacc[...] * (1.0 / hw)

        dn = (((1,), (1,)), ((), ()))
        h = jax.lax.dot_general(pooled, w1_ref[...], dn,
                                preferred_element_type=jnp.float32) + b1_ref[...]
        h = jnp.where(h >= 0, h, 0.2 * h)

        z = jax.lax.dot_general(pooled, w0_ref[...], dn,
                                preferred_element_type=jnp.float32) + b0_ref[...]
        noise = jnp.maximum(z, 0.0) + jnp.log1p(jnp.exp(-jnp.abs(z)))

        e = noise.shape[1]
        nmean = jnp.mean(noise, axis=1, keepdims=True)
        d = noise - nmean
        var = jnp.sum(d * d, axis=1, keepdims=True) * (1.0 / (e - 1))
        std = jnp.sqrt(var)
        std = jnp.where(std == 0, 1.0, std)
        scores = h + d / std

        iota = jax.lax.broadcasted_iota(jnp.int32, scores.shape, 1)
        work = scores
        mask = jnp.zeros_like(scores, dtype=jnp.bool_)
        for _ in range(_TOP_K):
            mx = jnp.max(work, axis=1, keepdims=True)
            first = jnp.min(jnp.where(work == mx, iota, e), axis=1, keepdims=True)
            sel = iota == first
            mask = jnp.logical_or(mask, sel)
            work = jnp.where(sel, -1e30, work)

        h_masked = jnp.where(mask, h, -1e9)
        hm = jnp.max(h_masked, axis=1, keepdims=True)
        ex = jnp.exp(h_masked - hm)
        out_ref[...] = ex / jnp.sum(ex, axis=1, keepdims=True)


@jax.jit
def kernel(x, W0, b0, W1, b1):
    B, C, H, W = x.shape
    E = W0.shape[0]
    hw = H * W
    hw_blk = 14
    n_steps = hw // hw_blk

    xp = x.transpose(2, 3, 0, 1).reshape(hw, B, C)

    out = pl.pallas_call(
        functools.partial(_gate_kernel, n_steps=n_steps, hw=float(hw)),
        grid=(n_steps,),
        in_specs=[
            pl.BlockSpec((hw_blk, B, C), lambda i: (i, 0, 0)),
            pl.BlockSpec((E, C), lambda i: (0, 0)),
            pl.BlockSpec((1, E), lambda i: (0, 0)),
            pl.BlockSpec((E, C), lambda i: (0, 0)),
            pl.BlockSpec((1, E), lambda i: (0, 0)),
        ],
        out_specs=pl.BlockSpec((B, E), lambda i: (0, 0)),
        out_shape=jax.ShapeDtypeStruct((B, E), jnp.float32),
        scratch_shapes=[
            pltpu.VMEM((B, C), jnp.float32),
            pltpu.VMEM((B, C), jnp.float32),
        ],
    )(xp, W0, b0.reshape(1, E), W1, b1.reshape(1, E))
    return out
